# baseline (speedup 1.0000x reference)
import functools

import jax
import jax.numpy as jnp
from jax import lax
from jax.experimental import pallas as pl
from jax.experimental.pallas import tpu as pltpu
from jax.experimental.pallas import tpu_sc as plsc

M = 50000
M_PAD = 50176
C = 32
E = 800000
NC = 2
NS = 16
CH = 128
SUB = 25
NO = 16
EPT = CH * SUB * NO
E_PAD = EPT * NS
ROWS_PT = M_PAD // NS
FB = 112
NF = ROWS_PT // FB


def _sc_powers(x_t, sl, dl, vl, su, du, vu):
    mesh = plsc.VectorSubcoreMesh(core_axis_name="c", subcore_axis_name="s")

    @functools.partial(
        pl.kernel,
        out_type=tuple(jax.ShapeDtypeStruct((M_PAD, C), jnp.float32)
                       for _ in range(4)),
        mesh=mesh,
        compiler_params=pltpu.CompilerParams(use_tc_tiling_on_sc=False),
        scratch_types=dict(
            acc=pltpu.VMEM_SHARED((M_PAD, C), jnp.float32),
            src_v=pltpu.VMEM((SUB, CH), jnp.int32),
            dst_v=pltpu.VMEM((SUB, CH), jnp.int32),
            val_v=pltpu.VMEM((SUB, CH), jnp.float32),
            rows_v=pltpu.VMEM((CH, C), jnp.float32),
            fbuf=pltpu.VMEM((FB, C), jnp.float32),
            zbuf=pltpu.VMEM((FB, C), jnp.float32),
            gsem=pltpu.SemaphoreType.DMA,
        ),
    )
    def run(x_hbm, sl_h, dl_h, vl_h, su_h, du_h, vu_h,
            y1l_h, y2l_h, y1u_h, y2u_h,
            acc, src_v, dst_v, val_v, rows_v, fbuf, zbuf, gsem):
        cid = lax.axis_index("c")
        sid = lax.axis_index("s")
        base = sid * ROWS_PT
        zv = jnp.zeros((16,), jnp.float32)

        def zfill(i, carry):
            zbuf[i, pl.ds(0, 16)] = zv
            zbuf[i, pl.ds(16, 16)] = zv
            return carry
        lax.fori_loop(0, FB, zfill, 0)

        def zero_acc(i, carry):
            pltpu.sync_copy(zbuf, acc.at[pl.ds(base + i * FB, FB)])
            return carry
        lax.fori_loop(0, NF, zero_acc, 0)
        plsc.subcore_barrier()

        def spmv(xsrc_ref, s_h, d_h, v_h):
            def outer(o, carry):
                pltpu.sync_copy(s_h.at[sid, o], src_v)
                pltpu.sync_copy(d_h.at[sid, o], dst_v)
                pltpu.sync_copy(v_h.at[sid, o], val_v)

                def chunk(j, carry2):
                    pltpu.async_copy(xsrc_ref.at[src_v.at[j]], rows_v,
                                     gsem).wait()

                    def scale(g, carry3):
                        vv = val_v[j, pl.ds(g * 16, 16)]
                        eb = g * 16
                        for e16 in range(16):
                            v = vv[e16]
                            rows_v[eb + e16, pl.ds(0, 16)] = (
                                rows_v[eb + e16, pl.ds(0, 16)] * v)
                            rows_v[eb + e16, pl.ds(16, 16)] = (
                                rows_v[eb + e16, pl.ds(16, 16)] * v)
                        return carry3
                    lax.fori_loop(0, CH // 16, scale, 0)
                    pltpu.sync_copy(rows_v, acc.at[dst_v.at[j]], add=True)
                    return carry2
                lax.fori_loop(0, SUB, chunk, 0)
                return carry
            lax.fori_loop(0, NO, outer, 0)

        def flush(dst_view):
            def fstep(i, carry):
                pltpu.sync_copy(acc.at[pl.ds(base + i * FB, FB)], fbuf)
                pltpu.sync_copy(fbuf, dst_view.at[pl.ds(base + i * FB, FB)])
                pltpu.sync_copy(zbuf, acc.at[pl.ds(base + i * FB, FB)])
                return carry
            lax.fori_loop(0, NF, fstep, 0)

        def chain(s_h, d_h, v_h, y1_h, y2_h):
            spmv(x_hbm, s_h, d_h, v_h)
            plsc.subcore_barrier()
            flush(y1_h)
            plsc.subcore_barrier()
            spmv(y1_h, s_h, d_h, v_h)
            plsc.subcore_barrier()
            flush(y2_h)

        @pl.when(cid == 0)
        def _():
            chain(sl_h, dl_h, vl_h, y1l_h, y2l_h)

        @pl.when(cid == 1)
        def _():
            chain(su_h, du_h, vu_h, y1u_h, y2u_h)

    return run(x_t, sl, dl, vl, su, du, vu)


MB = 7168
GRID_M = M_PAD // MB


def _tc_contract(W, x, feats, bias2):
    def body(w_ref, x_ref, f1, f2, f3, f4, b_ref, o_ref):
        acc = lax.dot_general(w_ref[0], x_ref[0], (((1,), (0,)), ((), ())),
                              preferred_element_type=jnp.float32)
        for k, fr in enumerate((f1, f2, f3, f4)):
            acc += lax.dot_general(w_ref[k + 1], fr[...],
                                   (((1,), (1,)), ((), ())),
                                   preferred_element_type=jnp.float32)
        o_ref[0] = acc + b_ref[...]

    fspec = pl.BlockSpec((MB, C), lambda i: (i, 0))
    return pl.pallas_call(
        body,
        grid=(GRID_M,),
        in_specs=[pl.BlockSpec((5, C, C), lambda i: (0, 0, 0)),
                  pl.BlockSpec((1, C, MB), lambda i: (0, 0, i)),
                  fspec, fspec, fspec, fspec,
                  pl.BlockSpec((C, 1), lambda i: (0, 0))],
        out_specs=pl.BlockSpec((1, C, MB), lambda i: (0, 0, i)),
        out_shape=jax.ShapeDtypeStruct((1, C, M), jnp.float32),
    )(W, x, *feats, bias2)


def kernel(Ll_indices, Ll_values, Lu_indices, Lu_values, x, theta, bias):
    x_t = jnp.pad(x[0].T, ((0, M_PAD - M), (0, 0)))

    def prep(ind, vals):
        pad = E_PAD - E
        s = jnp.pad(ind[1], (0, pad)).reshape(NS, NO, SUB, CH)
        d = jnp.pad(ind[0], (0, pad)).reshape(NS, NO, SUB, CH)
        v = jnp.pad(vals, (0, pad)).reshape(NS, NO, SUB, CH)
        return s, d, v

    sl, dl, vl = prep(Ll_indices, Ll_values)
    su, du, vu = prep(Lu_indices, Lu_values)

    y1l, y2l, y1u, y2u = _sc_powers(x_t, sl, dl, vl, su, du, vu)

    W = jnp.stack([theta[:, :, 0] + theta[:, :, 3], theta[:, :, 1],
                   theta[:, :, 2], theta[:, :, 4], theta[:, :, 5]])
    return _tc_contract(W, x, (y1l, y2l, y1u, y2u), bias[0])


# speedup vs baseline: 1.0060x; 1.0060x over previous
import functools

import jax
import jax.numpy as jnp
from jax import lax
from jax.experimental import pallas as pl
from jax.experimental.pallas import tpu as pltpu
from jax.experimental.pallas import tpu_sc as plsc

M = 50000
M_PAD = 50176
C = 32
E = 800000
NC = 2
NS = 16
CH = 128
SUB = 25
NO = 16
EPT = CH * SUB * NO
E_PAD = EPT * NS
ROWS_PT = M_PAD // NS
FB = 112
NF = ROWS_PT // FB


def _sc_powers(x_t, sl, dl, vl, su, du, vu):
    mesh = plsc.VectorSubcoreMesh(core_axis_name="c", subcore_axis_name="s")

    @functools.partial(
        pl.kernel,
        out_type=tuple(jax.ShapeDtypeStruct((M_PAD, C), jnp.float32)
                       for _ in range(4)),
        mesh=mesh,
        compiler_params=pltpu.CompilerParams(use_tc_tiling_on_sc=False),
        scratch_types=dict(
            acc=pltpu.VMEM_SHARED((M_PAD, C), jnp.float32),
            src_v=pltpu.VMEM((SUB, CH), jnp.int32),
            dst_v=pltpu.VMEM((SUB, CH), jnp.int32),
            val_v=pltpu.VMEM((SUB, CH), jnp.float32),
            rows_v=pltpu.VMEM((CH, C), jnp.float32),
            fbuf=pltpu.VMEM((FB, C), jnp.float32),
            zbuf=pltpu.VMEM((FB, C), jnp.float32),
            gsem=pltpu.SemaphoreType.DMA,
        ),
    )
    def run(x_hbm, sl_h, dl_h, vl_h, su_h, du_h, vu_h,
            y1l_h, y2l_h, y1u_h, y2u_h,
            acc, src_v, dst_v, val_v, rows_v, fbuf, zbuf, gsem):
        cid = lax.axis_index("c")
        sid = lax.axis_index("s")
        base = sid * ROWS_PT
        zv = jnp.zeros((16,), jnp.float32)

        def zfill(i, carry):
            zbuf[i, pl.ds(0, 16)] = zv
            zbuf[i, pl.ds(16, 16)] = zv
            return carry
        lax.fori_loop(0, FB, zfill, 0)

        def zero_acc(i, carry):
            pltpu.sync_copy(zbuf, acc.at[pl.ds(base + i * FB, FB)])
            return carry
        lax.fori_loop(0, NF, zero_acc, 0)
        plsc.subcore_barrier()

        def spmv(xsrc_ref, s_h, d_h, v_h):
            def outer(o, carry):
                pltpu.sync_copy(s_h.at[sid, o], src_v)
                pltpu.sync_copy(d_h.at[sid, o], dst_v)
                pltpu.sync_copy(v_h.at[sid, o], val_v)

                def chunk(j, carry2):
                    pltpu.async_copy(xsrc_ref.at[src_v.at[j]], rows_v,
                                     gsem).wait()

                    def scale(g, carry3):
                        vv = val_v[j, pl.ds(g * 16, 16)]
                        eb = g * 16
                        for e16 in range(16):
                            v = vv[e16]
                            rows_v[eb + e16, pl.ds(0, 16)] = (
                                rows_v[eb + e16, pl.ds(0, 16)] * v)
                            rows_v[eb + e16, pl.ds(16, 16)] = (
                                rows_v[eb + e16, pl.ds(16, 16)] * v)
                        return carry3
                    lax.fori_loop(0, CH // 16, scale, 0)
                    pltpu.sync_copy(rows_v, acc.at[dst_v.at[j]], add=True)
                    return carry2
                lax.fori_loop(0, SUB, chunk, 0)
                return carry
            lax.fori_loop(0, NO, outer, 0)

        def flush(dst_view):
            pltpu.sync_copy(acc.at[pl.ds(base, ROWS_PT)],
                            dst_view.at[pl.ds(base, ROWS_PT)])

            def fstep(i, carry):
                pltpu.sync_copy(zbuf, acc.at[pl.ds(base + i * FB, FB)])
                return carry
            lax.fori_loop(0, NF, fstep, 0)

        def chain(s_h, d_h, v_h, y1_h, y2_h):
            spmv(x_hbm, s_h, d_h, v_h)
            plsc.subcore_barrier()
            flush(y1_h)
            plsc.subcore_barrier()
            spmv(y1_h, s_h, d_h, v_h)
            plsc.subcore_barrier()
            flush(y2_h)

        @pl.when(cid == 0)
        def _():
            chain(sl_h, dl_h, vl_h, y1l_h, y2l_h)

        @pl.when(cid == 1)
        def _():
            chain(su_h, du_h, vu_h, y1u_h, y2u_h)

    return run(x_t, sl, dl, vl, su, du, vu)


MB = 7168
GRID_M = M_PAD // MB


def _tc_contract(W, x, feats, bias2):
    def body(w_ref, x_ref, f1, f2, f3, f4, b_ref, o_ref):
        acc = lax.dot_general(w_ref[0], x_ref[0], (((1,), (0,)), ((), ())),
                              preferred_element_type=jnp.float32)
        for k, fr in enumerate((f1, f2, f3, f4)):
            acc += lax.dot_general(w_ref[k + 1], fr[...],
                                   (((1,), (1,)), ((), ())),
                                   preferred_element_type=jnp.float32)
        o_ref[0] = acc + b_ref[...]

    fspec = pl.BlockSpec((MB, C), lambda i: (i, 0))
    return pl.pallas_call(
        body,
        grid=(GRID_M,),
        in_specs=[pl.BlockSpec((5, C, C), lambda i: (0, 0, 0)),
                  pl.BlockSpec((1, C, MB), lambda i: (0, 0, i)),
                  fspec, fspec, fspec, fspec,
                  pl.BlockSpec((C, 1), lambda i: (0, 0))],
        out_specs=pl.BlockSpec((1, C, MB), lambda i: (0, 0, i)),
        out_shape=jax.ShapeDtypeStruct((1, C, M), jnp.float32),
    )(W, x, *feats, bias2)


def kernel(Ll_indices, Ll_values, Lu_indices, Lu_values, x, theta, bias):
    x_t = jnp.pad(x[0].T, ((0, M_PAD - M), (0, 0)))

    def prep(ind, vals):
        pad = E_PAD - E
        s = jnp.pad(ind[1], (0, pad)).reshape(NS, NO, SUB, CH)
        d = jnp.pad(ind[0], (0, pad)).reshape(NS, NO, SUB, CH)
        v = jnp.pad(vals, (0, pad)).reshape(NS, NO, SUB, CH)
        return s, d, v

    sl, dl, vl = prep(Ll_indices, Ll_values)
    su, du, vu = prep(Lu_indices, Lu_values)

    y1l, y2l, y1u, y2u = _sc_powers(x_t, sl, dl, vl, su, du, vu)

    W = jnp.stack([theta[:, :, 0] + theta[:, :, 3], theta[:, :, 1],
                   theta[:, :, 2], theta[:, :, 4], theta[:, :, 5]])
    return _tc_contract(W, x, (y1l, y2l, y1u, y2u), bias[0])


# speedup vs baseline: 1.0098x; 1.0037x over previous
import functools

import jax
import jax.numpy as jnp
from jax import lax
from jax.experimental import pallas as pl
from jax.experimental.pallas import tpu as pltpu
from jax.experimental.pallas import tpu_sc as plsc

M = 50000
M_PAD = 50176
C = 32
E = 800000
NC = 2
NS = 16
CH = 128
SUB = 25
NO = 16
EPT = CH * SUB * NO
E_PAD = EPT * NS
ROWS_PT = M_PAD // NS
FB = 112
NF = ROWS_PT // FB


def _sc_powers(x_t, sl, dl, vl, su, du, vu):
    mesh = plsc.VectorSubcoreMesh(core_axis_name="c", subcore_axis_name="s")

    @functools.partial(
        pl.kernel,
        out_type=tuple(jax.ShapeDtypeStruct((M_PAD, C), jnp.float32)
                       for _ in range(4)),
        mesh=mesh,
        compiler_params=pltpu.CompilerParams(use_tc_tiling_on_sc=False),
        scratch_types=dict(
            acc=pltpu.VMEM_SHARED((M_PAD, C), jnp.float32),
            src_v=pltpu.VMEM((SUB, CH), jnp.int32),
            dst_v=pltpu.VMEM((SUB, CH), jnp.int32),
            val_v=pltpu.VMEM((SUB, CH), jnp.float32),
            rows_v=pltpu.VMEM((CH, C), jnp.float32),
            fbuf=pltpu.VMEM((FB, C), jnp.float32),
            zbuf=pltpu.VMEM((FB, C), jnp.float32),
            gsem=pltpu.SemaphoreType.DMA,
        ),
    )
    def run(x_hbm, sl_h, dl_h, vl_h, su_h, du_h, vu_h,
            y1l_h, y2l_h, y1u_h, y2u_h,
            acc, src_v, dst_v, val_v, rows_v, fbuf, zbuf, gsem):
        cid = lax.axis_index("c")
        sid = lax.axis_index("s")
        base = sid * ROWS_PT
        zv = jnp.zeros((16,), jnp.float32)

        def zfill(i, carry):
            zbuf[i, pl.ds(0, 16)] = zv
            zbuf[i, pl.ds(16, 16)] = zv
            return carry
        lax.fori_loop(0, FB, zfill, 0)

        def zero_acc(i, carry):
            pltpu.sync_copy(zbuf, acc.at[pl.ds(base + i * FB, FB)])
            return carry
        lax.fori_loop(0, NF, zero_acc, 0)
        plsc.subcore_barrier()

        def spmv(xsrc_ref, s_h, d_h, v_h):
            def outer(o, carry):
                pltpu.sync_copy(s_h.at[sid, o], src_v)
                pltpu.sync_copy(d_h.at[sid, o], dst_v)
                pltpu.sync_copy(v_h.at[sid, o], val_v)

                def chunk(j, carry2):
                    pltpu.async_copy(xsrc_ref.at[src_v.at[j]], rows_v,
                                     gsem).wait()

                    def scale(g, carry3):
                        vv = val_v[j, pl.ds(g * 16, 16)]
                        eb = g * 16
                        for e16 in range(16):
                            v = vv[e16]
                            rows_v[eb + e16, pl.ds(0, 16)] = (
                                rows_v[eb + e16, pl.ds(0, 16)] * v)
                            rows_v[eb + e16, pl.ds(16, 16)] = (
                                rows_v[eb + e16, pl.ds(16, 16)] * v)
                        return carry3
                    lax.fori_loop(0, CH // 16, scale, 0)
                    pltpu.sync_copy(rows_v, acc.at[dst_v.at[j]], add=True)
                    return carry2
                lax.fori_loop(0, SUB, chunk, 0)
                return carry
            lax.fori_loop(0, NO, outer, 0)

        def flush(dst_view, rezero):
            pltpu.sync_copy(acc.at[pl.ds(base, ROWS_PT)],
                            dst_view.at[pl.ds(base, ROWS_PT)])
            if rezero:
                def fstep(i, carry):
                    pltpu.sync_copy(zbuf, acc.at[pl.ds(base + i * FB, FB)])
                    return carry
                lax.fori_loop(0, NF, fstep, 0)

        def chain(s_h, d_h, v_h, y1_h, y2_h):
            spmv(x_hbm, s_h, d_h, v_h)
            plsc.subcore_barrier()
            flush(y1_h, rezero=True)
            plsc.subcore_barrier()
            spmv(y1_h, s_h, d_h, v_h)
            plsc.subcore_barrier()
            flush(y2_h, rezero=False)

        @pl.when(cid == 0)
        def _():
            chain(sl_h, dl_h, vl_h, y1l_h, y2l_h)

        @pl.when(cid == 1)
        def _():
            chain(su_h, du_h, vu_h, y1u_h, y2u_h)

    return run(x_t, sl, dl, vl, su, du, vu)


MB = 7168
GRID_M = M_PAD // MB


def _tc_contract(W, x, feats, bias2):
    def body(w_ref, x_ref, f1, f2, f3, f4, b_ref, o_ref):
        acc = lax.dot_general(w_ref[0], x_ref[0], (((1,), (0,)), ((), ())),
                              preferred_element_type=jnp.float32)
        for k, fr in enumerate((f1, f2, f3, f4)):
            acc += lax.dot_general(w_ref[k + 1], fr[...],
                                   (((1,), (1,)), ((), ())),
                                   preferred_element_type=jnp.float32)
        o_ref[0] = acc + b_ref[...]

    fspec = pl.BlockSpec((MB, C), lambda i: (i, 0))
    return pl.pallas_call(
        body,
        grid=(GRID_M,),
        in_specs=[pl.BlockSpec((5, C, C), lambda i: (0, 0, 0)),
                  pl.BlockSpec((1, C, MB), lambda i: (0, 0, i)),
                  fspec, fspec, fspec, fspec,
                  pl.BlockSpec((C, 1), lambda i: (0, 0))],
        out_specs=pl.BlockSpec((1, C, MB), lambda i: (0, 0, i)),
        out_shape=jax.ShapeDtypeStruct((1, C, M), jnp.float32),
    )(W, x, *feats, bias2)


def kernel(Ll_indices, Ll_values, Lu_indices, Lu_values, x, theta, bias):
    x_t = jnp.pad(x[0].T, ((0, M_PAD - M), (0, 0)))

    def prep(ind, vals):
        pad = E_PAD - E
        s = jnp.pad(ind[1], (0, pad)).reshape(NS, NO, SUB, CH)
        d = jnp.pad(ind[0], (0, pad)).reshape(NS, NO, SUB, CH)
        v = jnp.pad(vals, (0, pad)).reshape(NS, NO, SUB, CH)
        return s, d, v

    sl, dl, vl = prep(Ll_indices, Ll_values)
    su, du, vu = prep(Lu_indices, Lu_values)

    y1l, y2l, y1u, y2u = _sc_powers(x_t, sl, dl, vl, su, du, vu)

    W = jnp.stack([theta[:, :, 0] + theta[:, :, 3], theta[:, :, 1],
                   theta[:, :, 2], theta[:, :, 4], theta[:, :, 5]])
    return _tc_contract(W, x, (y1l, y2l, y1u, y2u), bias[0])
